# baseline (speedup 1.0000x reference)
import jax
import jax.numpy as jnp
from jax.experimental import pallas as pl
from jax.experimental.pallas import tpu as pltpu

_WALL_COT = 0.5
_NSEG = 8


def _stats_body(out_ref, in_ref, m_ref, s_ref, e_ref, d_ref,
                cnt_o, pos_o, err_o, recov_o):
    bb = pl.program_id(1)
    hc = pl.program_id(2)

    @pl.when(jnp.logical_and(bb == 0, hc == 0))
    def _init():
        cnt_o[...] = jnp.zeros_like(cnt_o)
        pos_o[...] = jnp.zeros_like(pos_o)
        err_o[...] = jnp.zeros_like(err_o)
        recov_o[...] = jnp.zeros_like(recov_o)

    m = m_ref[0, 0]
    o = out_ref[0]
    x = in_ref[0]
    t = jnp.where(m[None] >= _WALL_COT, 0.0, x)
    diff = o - t
    mse = jnp.sum(diff * diff, axis=0)
    mpos = m > 0.0
    recov_sum = jnp.sum(jnp.where(mpos, mse, 0.0), axis=0)
    recov_cnt = jnp.sum(mpos.astype(jnp.float32), axis=0)
    recov_o[0, 0:1, :] = recov_o[0, 0:1, :] + recov_sum[None]
    recov_o[0, 1:2, :] = recov_o[0, 1:2, :] + recov_cnt[None]

    e = e_ref[0]
    d = d_ref[0]
    ed = e - d
    re = jnp.sum(ed * ed, axis=0) / 96.0

    echunk, wechunk = e_ref.shape[2], e_ref.shape[3]
    hchunk, wchunk = m_ref.shape[2], m_ref.shape[3]
    he_i = jax.lax.broadcasted_iota(jnp.int32, (echunk, hchunk), 0)
    h_i = jax.lax.broadcasted_iota(jnp.int32, (echunk, hchunk), 1)
    p2 = (h_i == 4 * he_i).astype(jnp.float32)
    w_i = jax.lax.broadcasted_iota(jnp.int32, (wchunk, wechunk), 0)
    we_i = jax.lax.broadcasted_iota(jnp.int32, (wchunk, wechunk), 1)
    p1 = (w_i == 4 * we_i).astype(jnp.float32)

    seg = s_ref[0, 0]
    pm = jnp.logical_and(m < _WALL_COT, m > 0.0).astype(jnp.float32)
    seg_sub = jnp.dot(jnp.dot(p2, seg, preferred_element_type=jnp.float32),
                      p1, preferred_element_type=jnp.float32)
    pm_sub = jnp.dot(jnp.dot(p2, pm, preferred_element_type=jnp.float32),
                     p1, preferred_element_type=jnp.float32)

    cnt_rows = []
    pos_rows = []
    err_rows = []
    for s in range(_NSEG):
        ms = (seg_sub == float(s)).astype(jnp.float32)
        cnt_rows.append(jnp.sum(ms, axis=0)[None])
        pos_rows.append(jnp.sum(ms * pm_sub, axis=0)[None])
        err_rows.append(jnp.sum(ms * re, axis=0)[None])
    rows = pl.ds(bb * _NSEG, _NSEG)
    cnt_o[0, rows, :] = cnt_o[0, rows, :] + jnp.concatenate(cnt_rows, 0)
    pos_o[0, rows, :] = pos_o[0, rows, :] + jnp.concatenate(pos_rows, 0)
    err_o[0, rows, :] = err_o[0, rows, :] + jnp.concatenate(err_rows, 0)


def _combine_body(cnt_ref, pos_ref, err_ref, recov_ref, loss_ref):
    cnt = jnp.sum(cnt_ref[...], axis=2, keepdims=True)
    pos = jnp.sum(pos_ref[...], axis=2, keepdims=True)
    err = jnp.sum(err_ref[...], axis=2, keepdims=True)
    valid = jnp.logical_not(cnt / 16384.0 < 0.01)
    mean_err = err / cnt
    flags = jnp.logical_and(valid, pos / cnt > 0.01)
    pos_sum = jnp.sum(jnp.where(flags, mean_err, 0.0))
    pos_cnt = jnp.sum(flags.astype(jnp.float32))
    rs = jnp.sum(recov_ref[:, 0:1, :])
    rc = jnp.sum(recov_ref[:, 1:2, :])
    loss = rs / rc + pos_sum / pos_cnt
    loss_ref[...] = jnp.broadcast_to(loss, loss_ref.shape)


def kernel(outputs, inputs, enc1, dec1, masks, segs, confidence,
           iteration, epoch):
    B, C, H, W = outputs.shape
    _, Ce, He, We = enc1.shape
    ncore = 2
    bh = B // ncore
    nhc = 4
    hchunk = H // nhc
    echunk = He // nhc

    grid = (ncore, bh, nhc)
    f32 = jnp.float32
    cnt_o, pos_o, err_o, recov_o = pl.pallas_call(
        _stats_body,
        grid=grid,
        in_specs=[
            pl.BlockSpec((1, C, hchunk, W), lambda c, b, h: (c * bh + b, 0, h, 0)),
            pl.BlockSpec((1, C, hchunk, W), lambda c, b, h: (c * bh + b, 0, h, 0)),
            pl.BlockSpec((1, 1, hchunk, W), lambda c, b, h: (c * bh + b, 0, h, 0)),
            pl.BlockSpec((1, 1, hchunk, W), lambda c, b, h: (c * bh + b, 0, h, 0)),
            pl.BlockSpec((1, Ce, echunk, We), lambda c, b, h: (c * bh + b, 0, h, 0)),
            pl.BlockSpec((1, Ce, echunk, We), lambda c, b, h: (c * bh + b, 0, h, 0)),
        ],
        out_specs=[
            pl.BlockSpec((1, bh * _NSEG, We), lambda c, b, h: (c, 0, 0)),
            pl.BlockSpec((1, bh * _NSEG, We), lambda c, b, h: (c, 0, 0)),
            pl.BlockSpec((1, bh * _NSEG, We), lambda c, b, h: (c, 0, 0)),
            pl.BlockSpec((1, 8, W), lambda c, b, h: (c, 0, 0)),
        ],
        out_shape=[
            jax.ShapeDtypeStruct((ncore, bh * _NSEG, We), f32),
            jax.ShapeDtypeStruct((ncore, bh * _NSEG, We), f32),
            jax.ShapeDtypeStruct((ncore, bh * _NSEG, We), f32),
            jax.ShapeDtypeStruct((ncore, 8, W), f32),
        ],
        compiler_params=pltpu.CompilerParams(
            dimension_semantics=("parallel", "arbitrary", "arbitrary")),
    )(outputs, inputs, masks, segs, enc1, dec1)

    loss_out = pl.pallas_call(
        _combine_body,
        out_shape=jax.ShapeDtypeStruct((8, 128), f32),
    )(cnt_o, pos_o, err_o, recov_o)
    return loss_out[0, 0]
